# Initial kernel scaffold; baseline (speedup 1.0000x reference)
import jax
import jax.numpy as jnp
from jax.experimental import pallas as pl


def kernel(photo_one_hop, user_emb, item_emb, image_feats, text_feats, Q, K, V, W_onehop, W_mul1, W_mul2, edge_index, edge_weight):
    raise NotImplementedError("write your pallas kernel here")


# speedup vs baseline: 3.1649x; 3.1649x over previous
import functools

import jax
import jax.numpy as jnp
from jax import lax
from jax.experimental import pallas as pl
from jax.experimental.pallas import tpu as pltpu
from jax.experimental.pallas import tpu_sc as plsc

NUM_USERS = 25000
NUM_ITEMS = 25000
N = NUM_USERS + NUM_ITEMS
D = 64
N_LAYERS = 3
NH = 4
ATT = 16
L = 20
E = 800000

NPAD = 50176
ROWS_PER_TILE = NPAD // 16
EDGES_PER_TILE = E // 16
ECHUNK = 400
NCHUNK = EDGES_PER_TILE // ECHUNK

HOP_TOTAL = NUM_ITEMS * L
HOP_CHUNK = 1000
HOP_NCHUNK = HOP_TOTAL // HOP_CHUNK

def _hop_gather_body(idx_hbm, table_hbm, out_hbm, idx_v, rows_v, sem):
    c = lax.axis_index("c")
    s = lax.axis_index("s")
    wid = s * 2 + c
    for i in range(16):
        cid = wid + 32 * i

        @pl.when(cid < HOP_NCHUNK)
        def _():
            base = cid * HOP_CHUNK
            pltpu.sync_copy(idx_hbm.at[pl.ds(base, HOP_CHUNK)], idx_v)
            pltpu.async_copy(table_hbm.at[idx_v], rows_v, sem).wait()
            pltpu.sync_copy(rows_v, out_hbm.at[pl.ds(base, HOP_CHUNK)])


_RB = 200


def _dense_body(item_ref, hop_ref, img_ref, txt_ref, q_ref, k_ref, v_ref,
                wo_ref, w1_ref, w2_ref, out_ref):
    item = item_ref[...]
    hop = hop_ref[...]
    q = jnp.dot(item, q_ref[...], preferred_element_type=jnp.float32)
    hop2 = hop.reshape(_RB * L, D)
    k = jnp.dot(hop2, k_ref[...], preferred_element_type=jnp.float32)
    v = jnp.dot(hop2, v_ref[...], preferred_element_type=jnp.float32)
    k = k.reshape(_RB, L, D)
    v = v.reshape(_RB, L, D)

    p = q[:, None, :] * k
    t = p
    for sft in (1, 2, 4, 8):
        t = t + jnp.roll(t, -sft, axis=-1)
    lane = lax.broadcasted_iota(jnp.int32, (_RB, L, D), 2)
    t = jnp.where((lane % ATT) == 0, t, 0.0)
    for sft in (1, 2, 4, 8):
        t = t + jnp.roll(t, sft, axis=-1)
    s = t * 0.125
    m = jnp.max(s, axis=1, keepdims=True)
    e = jnp.exp(s - m)
    att = e / jnp.sum(e, axis=1, keepdims=True)
    mha = jnp.sum(att * v, axis=1)

    one_hop = jnp.dot(mha, wo_ref[...].T, preferred_element_type=jnp.float32)
    feats = jnp.concatenate([img_ref[...], txt_ref[...]], axis=-1)
    hidden = jnp.dot(feats, w1_ref[...].T, preferred_element_type=jnp.float32)
    itea = jnp.dot(hidden, w2_ref[...].T, preferred_element_type=jnp.float32)
    out_ref[...] = item * itea + one_hop


def _dense_stage(item_emb, hop_emb, image_feats, text_feats, Q, K, V,
                 W_onehop, W_mul1, W_mul2):
    grid = NUM_ITEMS // _RB
    full = lambda shp: pl.BlockSpec(shp, lambda i: (0,) * len(shp))
    return pl.pallas_call(
        _dense_body,
        grid=(grid,),
        in_specs=[
            pl.BlockSpec((_RB, D), lambda i: (i, 0)),
            pl.BlockSpec((_RB, L, D), lambda i: (i, 0, 0)),
            pl.BlockSpec((_RB, 128), lambda i: (i, 0)),
            pl.BlockSpec((_RB, 128), lambda i: (i, 0)),
            full((D, NH * ATT)),
            full((D, NH * ATT)),
            full((D, NH * ATT)),
            full((D, D)),
            full((4 * D, 256)),
            full((D, 4 * D)),
        ],
        out_specs=pl.BlockSpec((_RB, D), lambda i: (i, 0)),
        out_shape=jax.ShapeDtypeStruct((NUM_ITEMS, D), jnp.float32),
        compiler_params=pltpu.CompilerParams(
            dimension_semantics=("parallel",)),
    )(item_emb, hop_emb, image_feats, text_feats, Q, K, V,
      W_onehop, W_mul1, W_mul2)


def _spmm3_body(x0_hbm, src_hbm, dst_hbm, w_hbm, zrows_hbm,
           layers_hbm, mean_hbm, xbuf_hbm,
           acc, sidx, didx, wv, rows, m1, m2, m3, sem):
    c = lax.axis_index("c")
    s = lax.axis_index("s")
    HD = D // 2
    row0 = s * ROWS_PER_TILE
    ebase0 = s * EDGES_PER_TILE
    coff = c * NPAD

    for layer in range(N_LAYERS):
        xsrc = x0_hbm if layer == 0 else xbuf_hbm

        pltpu.sync_copy(zrows_hbm, acc.at[pl.ds(row0, ROWS_PER_TILE)])
        plsc.subcore_barrier()

        def chunk_body(i, _, xsrc=xsrc):
            base = ebase0 + i * ECHUNK
            pltpu.sync_copy(src_hbm.at[pl.ds(base, ECHUNK)], sidx)
            pltpu.sync_copy(dst_hbm.at[pl.ds(base, ECHUNK)], didx)
            pltpu.sync_copy(w_hbm.at[pl.ds(base, ECHUNK)], wv)

            def adj_body(k, _):
                sl = pl.ds(k * 16, 16)
                sidx[sl] = sidx[sl] + coff
                return 0
            lax.fori_loop(0, ECHUNK // 16, adj_body, 0, unroll=4)

            pltpu.async_copy(xsrc.at[sidx], rows, sem).wait()

            def grp_body(g, _):
                w16 = wv[pl.ds(g * 16, 16)]
                for kk in range(16):
                    j = g * 16 + kk
                    ws = lax.gather(
                        w16, jnp.full((16, 1), kk, jnp.int32),
                        lax.GatherDimensionNumbers(
                            offset_dims=(), collapsed_slice_dims=(0,),
                            start_index_map=(0,)),
                        (1,), mode=lax.GatherScatterMode.PROMISE_IN_BOUNDS)
                    a = rows[j, pl.ds(0, 16)] * ws
                    b = rows[j, pl.ds(16, 16)] * ws
                    rows[j, pl.ds(0, 16)] = a
                    rows[j, pl.ds(16, 16)] = b
                return 0
            lax.fori_loop(0, ECHUNK // 16, grp_body, 0)

            pltpu.sync_copy(rows, acc.at[didx], add=True)
            return 0

        lax.fori_loop(0, NCHUNK, chunk_body, 0)
        plsc.subcore_barrier()

        acc_sl = acc.at[pl.ds(row0, ROWS_PER_TILE)]
        pltpu.sync_copy(
            acc_sl,
            layers_hbm.at[pl.ds(row0, ROWS_PER_TILE), layer,
                          pl.ds(c * HD, HD)])
        if layer < N_LAYERS - 1:
            pltpu.sync_copy(
                acc_sl, xbuf_hbm.at[pl.ds(coff + row0, ROWS_PER_TILE)])
        else:
            for mc in range(28):
                r = row0 + mc * 112
                pltpu.sync_copy(
                    layers_hbm.at[pl.ds(r, 112), 0, pl.ds(c * HD, HD)], m1)
                pltpu.sync_copy(
                    layers_hbm.at[pl.ds(r, 112), 1, pl.ds(c * HD, HD)], m2)
                pltpu.sync_copy(acc.at[pl.ds(r, 112)], m3)

                def mean_body(j, _):
                    for h in range(2):
                        sl = pl.ds(h * 16, 16)
                        m3[j, sl] = (m1[j, sl] + m2[j, sl] + m3[j, sl]) * (
                            1.0 / 3.0)
                    return 0
                lax.fori_loop(0, 112, mean_body, 0, unroll=4)
                pltpu.sync_copy(
                    m3, mean_hbm.at[pl.ds(r, 112), pl.ds(c * HD, HD)])


@functools.lru_cache(maxsize=1)
def _sc_kernels():
    mesh = plsc.VectorSubcoreMesh(
        core_axis_name="c", subcore_axis_name="s",
        num_cores=2, num_subcores=16)
    sc_params = pltpu.CompilerParams(use_tc_tiling_on_sc=False)
    hop_gather = pl.kernel(
        _hop_gather_body,
        out_type=jax.ShapeDtypeStruct((HOP_TOTAL, D), jnp.float32),
        mesh=mesh,
        compiler_params=sc_params,
        scratch_types=[
            pltpu.VMEM((HOP_CHUNK,), jnp.int32),
            pltpu.VMEM((HOP_CHUNK, D), jnp.float32),
            pltpu.SemaphoreType.DMA,
        ],
    )
    spmm3 = pl.kernel(
        _spmm3_body,
        out_type=(
            jax.ShapeDtypeStruct((NPAD, N_LAYERS, D), jnp.float32),
            jax.ShapeDtypeStruct((NPAD, D), jnp.float32),
            jax.ShapeDtypeStruct((2 * NPAD, D // 2), jnp.float32),
        ),
        mesh=mesh,
        compiler_params=sc_params,
        scratch_types=[
            pltpu.VMEM_SHARED((NPAD, D // 2), jnp.float32),
            pltpu.VMEM((ECHUNK,), jnp.int32),
            pltpu.VMEM((ECHUNK,), jnp.int32),
            pltpu.VMEM((ECHUNK,), jnp.float32),
            pltpu.VMEM((ECHUNK, D // 2), jnp.float32),
            pltpu.VMEM((112, D // 2), jnp.float32),
            pltpu.VMEM((112, D // 2), jnp.float32),
            pltpu.VMEM((112, D // 2), jnp.float32),
            pltpu.SemaphoreType.DMA,
        ],
    )
    return hop_gather, spmm3


def kernel(photo_one_hop, user_emb, item_emb, image_feats, text_feats,
           Q, K, V, W_onehop, W_mul1, W_mul2, edge_index, edge_weight):
    hop_gather, spmm3 = _sc_kernels()
    hop_idx = photo_one_hop.reshape(-1).astype(jnp.int32)
    hop_emb = hop_gather(hop_idx, user_emb).reshape(NUM_ITEMS, L, D)

    all_items = _dense_stage(item_emb, hop_emb, image_feats, text_feats,
                             Q, K, V, W_onehop, W_mul1, W_mul2)

    ego = jnp.concatenate([user_emb, all_items], axis=0)
    ego = jnp.pad(ego, ((0, NPAD - N), (0, 0)))
    x0 = jnp.concatenate([ego[:, :D // 2], ego[:, D // 2:]], axis=0)
    zrows = jnp.zeros((ROWS_PER_TILE, D // 2), jnp.float32)

    src = edge_index[0].astype(jnp.int32)
    dst = edge_index[1].astype(jnp.int32)
    layers, mean, _ = spmm3(x0, src, dst, edge_weight, zrows)

    all_emb = layers[:N]
    mean_emb = mean[:N]
    return (mean_emb[:NUM_USERS], mean_emb[NUM_USERS:],
            all_emb[:NUM_USERS], all_emb[NUM_USERS:])


# speedup vs baseline: 4.3626x; 1.3784x over previous
import functools

import jax
import jax.numpy as jnp
from jax import lax
from jax.experimental import pallas as pl
from jax.experimental.pallas import tpu as pltpu
from jax.experimental.pallas import tpu_sc as plsc

NUM_USERS = 25000
NUM_ITEMS = 25000
N = NUM_USERS + NUM_ITEMS
D = 64
N_LAYERS = 3
NH = 4
ATT = 16
L = 20
E = 800000

NPAD = 50176
ROWS_PER_TILE = NPAD // 16
EDGES_PER_TILE = E // 16
ECHUNK = 400
NCHUNK = EDGES_PER_TILE // ECHUNK

HOP_TOTAL = NUM_ITEMS * L
HOP_CHUNK = 1000
HOP_NCHUNK = HOP_TOTAL // HOP_CHUNK

def _hop_gather_body(idx_hbm, table_hbm, out_hbm, idx_v, rows_v, sem):
    c = lax.axis_index("c")
    s = lax.axis_index("s")
    wid = s * 2 + c
    for i in range(16):
        cid = wid + 32 * i

        @pl.when(cid < HOP_NCHUNK)
        def _():
            base = cid * HOP_CHUNK
            pltpu.sync_copy(idx_hbm.at[pl.ds(base, HOP_CHUNK)], idx_v)
            pltpu.async_copy(table_hbm.at[idx_v], rows_v, sem).wait()
            pltpu.sync_copy(rows_v, out_hbm.at[pl.ds(base, HOP_CHUNK)])


_RB = 200


def _dense_body(item_ref, hop_ref, img_ref, txt_ref, q_ref, k_ref, v_ref,
                wo_ref, w1_ref, w2_ref, out_ref):
    item = item_ref[...]
    hop = hop_ref[...]
    q = jnp.dot(item, q_ref[...], preferred_element_type=jnp.float32)
    hop2 = hop.reshape(_RB * L, D)
    k = jnp.dot(hop2, k_ref[...], preferred_element_type=jnp.float32)
    v = jnp.dot(hop2, v_ref[...], preferred_element_type=jnp.float32)
    k = k.reshape(_RB, L, D)
    v = v.reshape(_RB, L, D)

    p = q[:, None, :] * k
    ai = lax.broadcasted_iota(jnp.int32, (D, D), 0) // ATT
    li = lax.broadcasted_iota(jnp.int32, (D, D), 1) // ATT
    mhead = jnp.where(ai == li, 0.125, 0.0)
    s = jnp.dot(p.reshape(_RB * L, D), mhead,
                preferred_element_type=jnp.float32).reshape(_RB, L, D)
    m = jnp.max(s, axis=1, keepdims=True)
    e = jnp.exp(s - m)
    att = e * (1.0 / jnp.sum(e, axis=1, keepdims=True))
    mha = jnp.sum(att * v, axis=1)

    one_hop = jnp.dot(mha, wo_ref[...].T, preferred_element_type=jnp.float32)
    feats = jnp.concatenate([img_ref[...], txt_ref[...]], axis=-1)
    hidden = jnp.dot(feats, w1_ref[...].T, preferred_element_type=jnp.float32)
    itea = jnp.dot(hidden, w2_ref[...].T, preferred_element_type=jnp.float32)
    out_ref[...] = item * itea + one_hop


def _dense_stage(item_emb, hop_emb, image_feats, text_feats, Q, K, V,
                 W_onehop, W_mul1, W_mul2):
    grid = NUM_ITEMS // _RB
    full = lambda shp: pl.BlockSpec(shp, lambda i: (0,) * len(shp))
    return pl.pallas_call(
        _dense_body,
        grid=(grid,),
        in_specs=[
            pl.BlockSpec((_RB, D), lambda i: (i, 0)),
            pl.BlockSpec((_RB, L, D), lambda i: (i, 0, 0)),
            pl.BlockSpec((_RB, 128), lambda i: (i, 0)),
            pl.BlockSpec((_RB, 128), lambda i: (i, 0)),
            full((D, NH * ATT)),
            full((D, NH * ATT)),
            full((D, NH * ATT)),
            full((D, D)),
            full((4 * D, 256)),
            full((D, 4 * D)),
        ],
        out_specs=pl.BlockSpec((_RB, D), lambda i: (i, 0)),
        out_shape=jax.ShapeDtypeStruct((NUM_ITEMS, D), jnp.float32),
        compiler_params=pltpu.CompilerParams(
            dimension_semantics=("parallel",)),
    )(item_emb, hop_emb, image_feats, text_feats, Q, K, V,
      W_onehop, W_mul1, W_mul2)


def _spmm3_body(x0_hbm, src_hbm, dst_hbm, w_hbm, zrows_hbm,
           layers_hbm, mean_hbm, xbuf_hbm,
           acc, sidx, didx, wv, rows, m1, m2, m3, sem):
    c = lax.axis_index("c")
    s = lax.axis_index("s")
    HD = D // 2
    row0 = s * ROWS_PER_TILE
    ebase0 = s * EDGES_PER_TILE
    coff = c * NPAD

    for layer in range(N_LAYERS):
        xsrc = x0_hbm if layer == 0 else xbuf_hbm

        pltpu.sync_copy(zrows_hbm, acc.at[pl.ds(row0, ROWS_PER_TILE)])
        plsc.subcore_barrier()

        def chunk_body(i, _, xsrc=xsrc):
            base = ebase0 + i * ECHUNK
            pltpu.sync_copy(src_hbm.at[pl.ds(base, ECHUNK)], sidx)
            pltpu.sync_copy(dst_hbm.at[pl.ds(base, ECHUNK)], didx)
            pltpu.sync_copy(w_hbm.at[pl.ds(base, ECHUNK)], wv)

            def adj_body(k, _):
                sl = pl.ds(k * 16, 16)
                sidx[sl] = sidx[sl] + coff
                return 0
            lax.fori_loop(0, ECHUNK // 16, adj_body, 0, unroll=4)

            pltpu.async_copy(xsrc.at[sidx], rows, sem).wait()

            def grp_body(g, _):
                w16 = wv[pl.ds(g * 16, 16)]
                for kk in range(16):
                    j = g * 16 + kk
                    ws = lax.gather(
                        w16, jnp.full((16, 1), kk, jnp.int32),
                        lax.GatherDimensionNumbers(
                            offset_dims=(), collapsed_slice_dims=(0,),
                            start_index_map=(0,)),
                        (1,), mode=lax.GatherScatterMode.PROMISE_IN_BOUNDS)
                    a = rows[j, pl.ds(0, 16)] * ws
                    b = rows[j, pl.ds(16, 16)] * ws
                    rows[j, pl.ds(0, 16)] = a
                    rows[j, pl.ds(16, 16)] = b
                return 0
            lax.fori_loop(0, ECHUNK // 16, grp_body, 0)

            pltpu.sync_copy(rows, acc.at[didx], add=True)
            return 0

        lax.fori_loop(0, NCHUNK, chunk_body, 0)
        plsc.subcore_barrier()

        acc_sl = acc.at[pl.ds(row0, ROWS_PER_TILE)]
        pltpu.sync_copy(
            acc_sl,
            layers_hbm.at[pl.ds(row0, ROWS_PER_TILE), layer,
                          pl.ds(c * HD, HD)])
        if layer < N_LAYERS - 1:
            pltpu.sync_copy(
                acc_sl, xbuf_hbm.at[pl.ds(coff + row0, ROWS_PER_TILE)])
        else:
            for mc in range(28):
                r = row0 + mc * 112
                pltpu.sync_copy(
                    layers_hbm.at[pl.ds(r, 112), 0, pl.ds(c * HD, HD)], m1)
                pltpu.sync_copy(
                    layers_hbm.at[pl.ds(r, 112), 1, pl.ds(c * HD, HD)], m2)
                pltpu.sync_copy(acc.at[pl.ds(r, 112)], m3)

                def mean_body(j, _):
                    for h in range(2):
                        sl = pl.ds(h * 16, 16)
                        m3[j, sl] = (m1[j, sl] + m2[j, sl] + m3[j, sl]) * (
                            1.0 / 3.0)
                    return 0
                lax.fori_loop(0, 112, mean_body, 0, unroll=4)
                pltpu.sync_copy(
                    m3, mean_hbm.at[pl.ds(r, 112), pl.ds(c * HD, HD)])


@functools.lru_cache(maxsize=1)
def _sc_kernels():
    mesh = plsc.VectorSubcoreMesh(
        core_axis_name="c", subcore_axis_name="s",
        num_cores=2, num_subcores=16)
    sc_params = pltpu.CompilerParams(use_tc_tiling_on_sc=False)
    hop_gather = pl.kernel(
        _hop_gather_body,
        out_type=jax.ShapeDtypeStruct((HOP_TOTAL, D), jnp.float32),
        mesh=mesh,
        compiler_params=sc_params,
        scratch_types=[
            pltpu.VMEM((HOP_CHUNK,), jnp.int32),
            pltpu.VMEM((HOP_CHUNK, D), jnp.float32),
            pltpu.SemaphoreType.DMA,
        ],
    )
    spmm3 = pl.kernel(
        _spmm3_body,
        out_type=(
            jax.ShapeDtypeStruct((NPAD, N_LAYERS, D), jnp.float32),
            jax.ShapeDtypeStruct((NPAD, D), jnp.float32),
            jax.ShapeDtypeStruct((2 * NPAD, D // 2), jnp.float32),
        ),
        mesh=mesh,
        compiler_params=sc_params,
        scratch_types=[
            pltpu.VMEM_SHARED((NPAD, D // 2), jnp.float32),
            pltpu.VMEM((ECHUNK,), jnp.int32),
            pltpu.VMEM((ECHUNK,), jnp.int32),
            pltpu.VMEM((ECHUNK,), jnp.float32),
            pltpu.VMEM((ECHUNK, D // 2), jnp.float32),
            pltpu.VMEM((112, D // 2), jnp.float32),
            pltpu.VMEM((112, D // 2), jnp.float32),
            pltpu.VMEM((112, D // 2), jnp.float32),
            pltpu.SemaphoreType.DMA,
        ],
    )
    return hop_gather, spmm3


def kernel(photo_one_hop, user_emb, item_emb, image_feats, text_feats,
           Q, K, V, W_onehop, W_mul1, W_mul2, edge_index, edge_weight):
    hop_gather, spmm3 = _sc_kernels()
    hop_idx = photo_one_hop.reshape(-1).astype(jnp.int32)
    hop_emb = hop_gather(hop_idx, user_emb).reshape(NUM_ITEMS, L, D)

    all_items = _dense_stage(item_emb, hop_emb, image_feats, text_feats,
                             Q, K, V, W_onehop, W_mul1, W_mul2)

    ego = jnp.concatenate([user_emb, all_items], axis=0)
    ego = jnp.pad(ego, ((0, NPAD - N), (0, 0)))
    x0 = jnp.concatenate([ego[:, :D // 2], ego[:, D // 2:]], axis=0)
    zrows = jnp.zeros((ROWS_PER_TILE, D // 2), jnp.float32)

    src = edge_index[0].astype(jnp.int32)
    dst = edge_index[1].astype(jnp.int32)
    layers, mean, _ = spmm3(x0, src, dst, edge_weight, zrows)

    all_emb = layers[:N]
    mean_emb = mean[:N]
    return (mean_emb[:NUM_USERS], mean_emb[NUM_USERS:],
            all_emb[:NUM_USERS], all_emb[NUM_USERS:])


# speedup vs baseline: 5.9437x; 1.3624x over previous
import functools

import jax
import jax.numpy as jnp
from jax import lax
from jax.experimental import pallas as pl
from jax.experimental.pallas import tpu as pltpu
from jax.experimental.pallas import tpu_sc as plsc

NUM_USERS = 25000
NUM_ITEMS = 25000
N = NUM_USERS + NUM_ITEMS
D = 64
N_LAYERS = 3
NH = 4
ATT = 16
L = 20
E = 800000

NPAD = 50176
ROWS_PER_TILE = NPAD // 16
EDGES_PER_TILE = E // 16
ECHUNK = 400
NCHUNK = EDGES_PER_TILE // ECHUNK

HOP_TOTAL = NUM_ITEMS * L
HOP_CHUNK = 1000
HOP_NCHUNK = HOP_TOTAL // HOP_CHUNK

def _hop_gather_body(idx_hbm, table_hbm, out_hbm, idx_v, rows_v, sem):
    c = lax.axis_index("c")
    s = lax.axis_index("s")
    wid = s * 2 + c
    for i in range(16):
        cid = wid + 32 * i

        @pl.when(cid < HOP_NCHUNK)
        def _():
            base = cid * HOP_CHUNK
            pltpu.sync_copy(idx_hbm.at[pl.ds(base, HOP_CHUNK)], idx_v)
            pltpu.async_copy(table_hbm.at[idx_v], rows_v, sem).wait()
            pltpu.sync_copy(rows_v, out_hbm.at[pl.ds(base, HOP_CHUNK)])


_RB = 200


def _dense_body(item_ref, hop_ref, img_ref, txt_ref, q_ref, k_ref, v_ref,
                wo_ref, w1_ref, w2_ref, out_ref):
    item = item_ref[...]
    hop = hop_ref[...]
    q = jnp.dot(item, q_ref[...], preferred_element_type=jnp.float32)
    hop2 = hop.reshape(_RB * L, D)
    k = jnp.dot(hop2, k_ref[...], preferred_element_type=jnp.float32)
    v = jnp.dot(hop2, v_ref[...], preferred_element_type=jnp.float32)
    k = k.reshape(_RB, L, D)
    v = v.reshape(_RB, L, D)

    p = q[:, None, :] * k
    ai = lax.broadcasted_iota(jnp.int32, (D, D), 0) // ATT
    li = lax.broadcasted_iota(jnp.int32, (D, D), 1) // ATT
    mhead = jnp.where(ai == li, 0.125, 0.0)
    s = jnp.dot(p.reshape(_RB * L, D), mhead,
                preferred_element_type=jnp.float32).reshape(_RB, L, D)
    m = jnp.max(s, axis=1, keepdims=True)
    e = jnp.exp(s - m)
    att = e * (1.0 / jnp.sum(e, axis=1, keepdims=True))
    mha = jnp.sum(att * v, axis=1)

    one_hop = jnp.dot(mha, wo_ref[...].T, preferred_element_type=jnp.float32)
    feats = jnp.concatenate([img_ref[...], txt_ref[...]], axis=-1)
    hidden = jnp.dot(feats, w1_ref[...].T, preferred_element_type=jnp.float32)
    itea = jnp.dot(hidden, w2_ref[...].T, preferred_element_type=jnp.float32)
    out_ref[...] = item * itea + one_hop


def _dense_stage(item_emb, hop_emb, image_feats, text_feats, Q, K, V,
                 W_onehop, W_mul1, W_mul2):
    grid = NUM_ITEMS // _RB
    full = lambda shp: pl.BlockSpec(shp, lambda i: (0,) * len(shp))
    return pl.pallas_call(
        _dense_body,
        grid=(grid,),
        in_specs=[
            pl.BlockSpec((_RB, D), lambda i: (i, 0)),
            pl.BlockSpec((_RB, L, D), lambda i: (i, 0, 0)),
            pl.BlockSpec((_RB, 128), lambda i: (i, 0)),
            pl.BlockSpec((_RB, 128), lambda i: (i, 0)),
            full((D, NH * ATT)),
            full((D, NH * ATT)),
            full((D, NH * ATT)),
            full((D, D)),
            full((4 * D, 256)),
            full((D, 4 * D)),
        ],
        out_specs=pl.BlockSpec((_RB, D), lambda i: (i, 0)),
        out_shape=jax.ShapeDtypeStruct((NUM_ITEMS, D), jnp.float32),
        compiler_params=pltpu.CompilerParams(
            dimension_semantics=("parallel",)),
    )(item_emb, hop_emb, image_feats, text_feats, Q, K, V,
      W_onehop, W_mul1, W_mul2)


def _scale_rows(rows, wv):
    def grp_body(g, _):
        w16 = wv[pl.ds(g * 16, 16)]
        for kk in range(16):
            j = g * 16 + kk
            ws = lax.gather(
                w16, jnp.full((16, 1), kk, jnp.int32),
                lax.GatherDimensionNumbers(
                    offset_dims=(), collapsed_slice_dims=(0,),
                    start_index_map=(0,)),
                (1,), mode=lax.GatherScatterMode.PROMISE_IN_BOUNDS)
            a = rows[j, pl.ds(0, 16)] * ws
            b = rows[j, pl.ds(16, 16)] * ws
            rows[j, pl.ds(0, 16)] = a
            rows[j, pl.ds(16, 16)] = b
        return 0
    lax.fori_loop(0, ECHUNK // 16, grp_body, 0)


def _spmm3_body(x0_hbm, src2_hbm, dst_hbm, w_hbm, zrows_hbm,
                layers_hbm, mean_hbm, xbuf_hbm,
                acc, sidx0, sidx1, didx0, didx1, wv0, wv1, rows0, rows1,
                si0, si1, sg0, sg1, ss0, ss1):
    c = lax.axis_index("c")
    s = lax.axis_index("s")
    HD = D // 2
    row0 = s * ROWS_PER_TILE
    ebase0 = s * EDGES_PER_TILE

    def start_idx(i, sidx, didx, wv, sem):
        base = ebase0 + i * ECHUNK
        pltpu.async_copy(src2_hbm.at[c, pl.ds(base, ECHUNK)], sidx, sem)
        pltpu.async_copy(dst_hbm.at[pl.ds(base, ECHUNK)], didx, sem)
        pltpu.async_copy(w_hbm.at[pl.ds(base, ECHUNK)], wv, sem)

    def wait_idx(sidx, didx, wv, sem):
        pltpu.make_async_copy(src2_hbm.at[c, pl.ds(0, ECHUNK)], sidx,
                              sem).wait()
        pltpu.make_async_copy(dst_hbm.at[pl.ds(0, ECHUNK)], didx, sem).wait()
        pltpu.make_async_copy(w_hbm.at[pl.ds(0, ECHUNK)], wv, sem).wait()

    def wait_scatter(rows, didx, sem):
        pltpu.make_async_copy(rows, acc.at[didx], sem).wait()

    for layer in range(N_LAYERS):
        xsrc = x0_hbm if layer == 0 else xbuf_hbm

        pltpu.sync_copy(zrows_hbm, acc.at[pl.ds(row0, ROWS_PER_TILE)])
        plsc.subcore_barrier()

        start_idx(0, sidx0, didx0, wv0, si0)
        wait_idx(sidx0, didx0, wv0, si0)
        hg = pltpu.async_copy(xsrc.at[sidx0], rows0, sg0)
        start_idx(1, sidx1, didx1, wv1, si1)
        hg.wait()
        _scale_rows(rows0, wv0)
        pltpu.async_copy(rows0, acc.at[didx0], ss0, add=True)

        def loop_body(g, _, xsrc=xsrc):
            wait_idx(sidx1, didx1, wv1, si1)
            ha = pltpu.async_copy(xsrc.at[sidx1], rows1, sg1)
            wait_scatter(rows0, didx0, ss0)
            start_idx(jnp.minimum(2 * g + 2, NCHUNK - 1),
                      sidx0, didx0, wv0, si0)
            ha.wait()
            _scale_rows(rows1, wv1)
            pltpu.async_copy(rows1, acc.at[didx1], ss1, add=True)
            wait_idx(sidx0, didx0, wv0, si0)
            hb = pltpu.async_copy(xsrc.at[sidx0], rows0, sg0)
            wait_scatter(rows1, didx1, ss1)
            start_idx(jnp.minimum(2 * g + 3, NCHUNK - 1),
                      sidx1, didx1, wv1, si1)
            hb.wait()
            _scale_rows(rows0, wv0)
            pltpu.async_copy(rows0, acc.at[didx0], ss0, add=True)
            return 0

        lax.fori_loop(0, (NCHUNK - 1) // 2, loop_body, 0)
        wait_scatter(rows0, didx0, ss0)
        wait_idx(sidx1, didx1, wv1, si1)
        plsc.subcore_barrier()

        acc_sl = acc.at[pl.ds(row0, ROWS_PER_TILE)]
        pltpu.sync_copy(
            acc_sl,
            layers_hbm.at[pl.ds(row0, ROWS_PER_TILE), layer,
                          pl.ds(c * HD, HD)])
        if layer < N_LAYERS - 1:
            pltpu.sync_copy(
                acc_sl, xbuf_hbm.at[pl.ds(c * NPAD + row0, ROWS_PER_TILE)])
        else:
            for mc in range(8):
                r = row0 + mc * 392
                pltpu.sync_copy(
                    layers_hbm.at[pl.ds(r, 392), 0, pl.ds(c * HD, HD)],
                    rows0.at[pl.ds(0, 392)])
                pltpu.sync_copy(acc.at[pl.ds(r, 392)],
                                rows1.at[pl.ds(0, 392)])

                def add_body(j, _):
                    for h in range(2):
                        sl = pl.ds(h * 16, 16)
                        rows1[j, sl] = rows1[j, sl] + rows0[j, sl]
                    return 0
                lax.fori_loop(0, 392, add_body, 0, unroll=4)
                pltpu.sync_copy(
                    layers_hbm.at[pl.ds(r, 392), 1, pl.ds(c * HD, HD)],
                    rows0.at[pl.ds(0, 392)])

                def fin_body(j, _):
                    for h in range(2):
                        sl = pl.ds(h * 16, 16)
                        rows1[j, sl] = (rows1[j, sl] + rows0[j, sl]) * (
                            1.0 / 3.0)
                    return 0
                lax.fori_loop(0, 392, fin_body, 0, unroll=4)
                pltpu.sync_copy(
                    rows1.at[pl.ds(0, 392)],
                    mean_hbm.at[pl.ds(r, 392), pl.ds(c * HD, HD)])


@functools.lru_cache(maxsize=1)
def _sc_kernels():
    mesh = plsc.VectorSubcoreMesh(
        core_axis_name="c", subcore_axis_name="s",
        num_cores=2, num_subcores=16)
    sc_params = pltpu.CompilerParams(use_tc_tiling_on_sc=False)
    hop_gather = pl.kernel(
        _hop_gather_body,
        out_type=jax.ShapeDtypeStruct((HOP_TOTAL, D), jnp.float32),
        mesh=mesh,
        compiler_params=sc_params,
        scratch_types=[
            pltpu.VMEM((HOP_CHUNK,), jnp.int32),
            pltpu.VMEM((HOP_CHUNK, D), jnp.float32),
            pltpu.SemaphoreType.DMA,
        ],
    )
    spmm3 = pl.kernel(
        _spmm3_body,
        out_type=(
            jax.ShapeDtypeStruct((NPAD, N_LAYERS, D), jnp.float32),
            jax.ShapeDtypeStruct((NPAD, D), jnp.float32),
            jax.ShapeDtypeStruct((2 * NPAD, D // 2), jnp.float32),
        ),
        mesh=mesh,
        compiler_params=sc_params,
        scratch_types=[
            pltpu.VMEM_SHARED((NPAD, D // 2), jnp.float32),
            pltpu.VMEM((ECHUNK,), jnp.int32),
            pltpu.VMEM((ECHUNK,), jnp.int32),
            pltpu.VMEM((ECHUNK,), jnp.int32),
            pltpu.VMEM((ECHUNK,), jnp.int32),
            pltpu.VMEM((ECHUNK,), jnp.float32),
            pltpu.VMEM((ECHUNK,), jnp.float32),
            pltpu.VMEM((ECHUNK, D // 2), jnp.float32),
            pltpu.VMEM((ECHUNK, D // 2), jnp.float32),
            pltpu.SemaphoreType.DMA,
            pltpu.SemaphoreType.DMA,
            pltpu.SemaphoreType.DMA,
            pltpu.SemaphoreType.DMA,
            pltpu.SemaphoreType.DMA,
            pltpu.SemaphoreType.DMA,
        ],
    )
    return hop_gather, spmm3


def kernel(photo_one_hop, user_emb, item_emb, image_feats, text_feats,
           Q, K, V, W_onehop, W_mul1, W_mul2, edge_index, edge_weight):
    hop_gather, spmm3 = _sc_kernels()
    hop_idx = photo_one_hop.reshape(-1).astype(jnp.int32)
    hop_emb = hop_gather(hop_idx, user_emb).reshape(NUM_ITEMS, L, D)

    all_items = _dense_stage(item_emb, hop_emb, image_feats, text_feats,
                             Q, K, V, W_onehop, W_mul1, W_mul2)

    ego = jnp.concatenate([user_emb, all_items], axis=0)
    ego = jnp.pad(ego, ((0, NPAD - N), (0, 0)))
    x0 = jnp.concatenate([ego[:, :D // 2], ego[:, D // 2:]], axis=0)
    zrows = jnp.zeros((ROWS_PER_TILE, D // 2), jnp.float32)

    src = edge_index[0].astype(jnp.int32)
    dst = edge_index[1].astype(jnp.int32)
    src2 = jnp.stack([src, src + NPAD], axis=0)
    layers, mean, _ = spmm3(x0, src2, dst, edge_weight, zrows)

    all_emb = layers[:N]
    mean_emb = mean[:N]
    return (mean_emb[:NUM_USERS], mean_emb[NUM_USERS:],
            all_emb[:NUM_USERS], all_emb[NUM_USERS:])


# speedup vs baseline: 6.5602x; 1.1037x over previous
import functools

import jax
import jax.numpy as jnp
from jax import lax
from jax.experimental import pallas as pl
from jax.experimental.pallas import tpu as pltpu
from jax.experimental.pallas import tpu_sc as plsc

NUM_USERS = 25000
NUM_ITEMS = 25000
N = NUM_USERS + NUM_ITEMS
D = 64
N_LAYERS = 3
NH = 4
ATT = 16
L = 20
E = 800000

NPAD = 50000
ROWS_PER_TILE = NPAD // 16
EDGES_PER_TILE = E // 16
ECHUNK = 400
NCHUNK = EDGES_PER_TILE // ECHUNK

HOP_TOTAL = NUM_ITEMS * L
HOP_CHUNK = 1000
HOP_NCHUNK = HOP_TOTAL // HOP_CHUNK

def _hop_gather_body(idx_hbm, table_hbm, out_hbm, idx_v, rows_v, sem):
    c = lax.axis_index("c")
    s = lax.axis_index("s")
    wid = s * 2 + c
    for i in range(16):
        cid = wid + 32 * i

        @pl.when(cid < HOP_NCHUNK)
        def _():
            base = cid * HOP_CHUNK
            pltpu.sync_copy(idx_hbm.at[pl.ds(base, HOP_CHUNK)], idx_v)
            pltpu.async_copy(table_hbm.at[idx_v], rows_v, sem).wait()
            pltpu.sync_copy(rows_v, out_hbm.at[pl.ds(base, HOP_CHUNK)])


_RB = 200


def _dense_body(item_ref, hop_ref, img_ref, txt_ref, q_ref, k_ref, v_ref,
                wo_ref, w1_ref, w2_ref, out_ref):
    item = item_ref[...]
    hop = hop_ref[...]
    q = jnp.dot(item, q_ref[...], preferred_element_type=jnp.float32)
    hop2 = hop
    k = jnp.dot(hop2, k_ref[...], preferred_element_type=jnp.float32)
    v = jnp.dot(hop2, v_ref[...], preferred_element_type=jnp.float32)
    k = k.reshape(_RB, L, D)
    v = v.reshape(_RB, L, D)

    p = q[:, None, :] * k
    ai = lax.broadcasted_iota(jnp.int32, (D, D), 0) // ATT
    li = lax.broadcasted_iota(jnp.int32, (D, D), 1) // ATT
    mhead = jnp.where(ai == li, 0.125, 0.0)
    s = jnp.dot(p.reshape(_RB * L, D), mhead,
                preferred_element_type=jnp.float32).reshape(_RB, L, D)
    m = jnp.max(s, axis=1, keepdims=True)
    e = jnp.exp(s - m)
    att = e * (1.0 / jnp.sum(e, axis=1, keepdims=True))
    mha = jnp.sum(att * v, axis=1)

    one_hop = jnp.dot(mha, wo_ref[...].T, preferred_element_type=jnp.float32)
    feats = jnp.concatenate([img_ref[...], txt_ref[...]], axis=-1)
    hidden = jnp.dot(feats, w1_ref[...].T, preferred_element_type=jnp.float32)
    itea = jnp.dot(hidden, w2_ref[...].T, preferred_element_type=jnp.float32)
    out_ref[...] = item * itea + one_hop


def _dense_stage(item_emb, hop_emb, image_feats, text_feats, Q, K, V,
                 W_onehop, W_mul1, W_mul2):
    grid = NUM_ITEMS // _RB
    full = lambda shp: pl.BlockSpec(shp, lambda i: (0,) * len(shp))
    return pl.pallas_call(
        _dense_body,
        grid=(grid,),
        in_specs=[
            pl.BlockSpec((_RB, D), lambda i: (i, 0)),
            pl.BlockSpec((_RB * L, D), lambda i: (i, 0)),
            pl.BlockSpec((_RB, 128), lambda i: (i, 0)),
            pl.BlockSpec((_RB, 128), lambda i: (i, 0)),
            full((D, NH * ATT)),
            full((D, NH * ATT)),
            full((D, NH * ATT)),
            full((D, D)),
            full((4 * D, 256)),
            full((D, 4 * D)),
        ],
        out_specs=pl.BlockSpec((_RB, D), lambda i: (i, 0)),
        out_shape=jax.ShapeDtypeStruct((NUM_ITEMS, D), jnp.float32),
        compiler_params=pltpu.CompilerParams(
            dimension_semantics=("parallel",)),
    )(item_emb, hop_emb, image_feats, text_feats, Q, K, V,
      W_onehop, W_mul1, W_mul2)


def _scale_rows(rows, wv):
    def grp_body(g, _):
        w16 = wv[pl.ds(g * 16, 16)]
        for kk in range(16):
            j = g * 16 + kk
            ws = lax.gather(
                w16, jnp.full((16, 1), kk, jnp.int32),
                lax.GatherDimensionNumbers(
                    offset_dims=(), collapsed_slice_dims=(0,),
                    start_index_map=(0,)),
                (1,), mode=lax.GatherScatterMode.PROMISE_IN_BOUNDS)
            a = rows[j, pl.ds(0, 16)] * ws
            b = rows[j, pl.ds(16, 16)] * ws
            rows[j, pl.ds(0, 16)] = a
            rows[j, pl.ds(16, 16)] = b
        return 0
    lax.fori_loop(0, ECHUNK // 16, grp_body, 0)


def _spmm3_body(x0_hbm, src2_hbm, dst_hbm, w_hbm, zrows_hbm,
                ulay_hbm, ilay_hbm, umean_hbm, imean_hbm, xbuf_hbm,
                acc, sidx0, sidx1, didx0, didx1, wv0, wv1, rows0, rows1,
                si0, si1, sg0, sg1, ss0, ss1):
    c = lax.axis_index("c")
    s = lax.axis_index("s")
    HD = D // 2
    row0 = s * ROWS_PER_TILE
    ebase0 = s * EDGES_PER_TILE

    def start_idx(i, sidx, didx, wv, sem):
        base = ebase0 + i * ECHUNK
        pltpu.async_copy(src2_hbm.at[c, pl.ds(base, ECHUNK)], sidx, sem)
        pltpu.async_copy(dst_hbm.at[pl.ds(base, ECHUNK)], didx, sem)
        pltpu.async_copy(w_hbm.at[pl.ds(base, ECHUNK)], wv, sem)

    def wait_idx(sidx, didx, wv, sem):
        pltpu.make_async_copy(src2_hbm.at[c, pl.ds(0, ECHUNK)], sidx,
                              sem).wait()
        pltpu.make_async_copy(dst_hbm.at[pl.ds(0, ECHUNK)], didx, sem).wait()
        pltpu.make_async_copy(w_hbm.at[pl.ds(0, ECHUNK)], wv, sem).wait()

    def wait_scatter(rows, didx, sem):
        pltpu.make_async_copy(rows, acc.at[didx], sem).wait()

    for layer in range(N_LAYERS):
        xsrc = x0_hbm if layer == 0 else xbuf_hbm

        pltpu.sync_copy(zrows_hbm, acc.at[pl.ds(row0, ROWS_PER_TILE)])
        plsc.subcore_barrier()

        start_idx(0, sidx0, didx0, wv0, si0)
        wait_idx(sidx0, didx0, wv0, si0)
        hg = pltpu.async_copy(xsrc.at[sidx0], rows0, sg0)
        start_idx(1, sidx1, didx1, wv1, si1)
        hg.wait()
        _scale_rows(rows0, wv0)
        pltpu.async_copy(rows0, acc.at[didx0], ss0, add=True)

        def loop_body(g, _, xsrc=xsrc):
            wait_idx(sidx1, didx1, wv1, si1)
            ha = pltpu.async_copy(xsrc.at[sidx1], rows1, sg1)
            wait_scatter(rows0, didx0, ss0)
            start_idx(jnp.minimum(2 * g + 2, NCHUNK - 1),
                      sidx0, didx0, wv0, si0)
            ha.wait()
            _scale_rows(rows1, wv1)
            pltpu.async_copy(rows1, acc.at[didx1], ss1, add=True)
            wait_idx(sidx0, didx0, wv0, si0)
            hb = pltpu.async_copy(xsrc.at[sidx0], rows0, sg0)
            wait_scatter(rows1, didx1, ss1)
            start_idx(jnp.minimum(2 * g + 3, NCHUNK - 1),
                      sidx1, didx1, wv1, si1)
            hb.wait()
            _scale_rows(rows0, wv0)
            pltpu.async_copy(rows0, acc.at[didx0], ss0, add=True)
            return 0

        lax.fori_loop(0, (NCHUNK - 1) // 2, loop_body, 0)
        wait_scatter(rows0, didx0, ss0)
        wait_idx(sidx1, didx1, wv1, si1)
        plsc.subcore_barrier()

        acc_sl = acc.at[pl.ds(row0, ROWS_PER_TILE)]
        lrow0 = jnp.where(s >= 8, row0 - NUM_USERS, row0)

        def flush_half(lay_hbm, mean_hbm):
            pltpu.sync_copy(
                acc_sl,
                lay_hbm.at[pl.ds(lrow0, ROWS_PER_TILE), layer,
                           pl.ds(c * HD, HD)])
            if layer == N_LAYERS - 1:
                def mean_chunk(mc, _):
                    r = lrow0 + mc * 125
                    pltpu.sync_copy(
                        lay_hbm.at[pl.ds(r, 125), 0, pl.ds(c * HD, HD)],
                        rows0.at[pl.ds(0, 125)])
                    pltpu.sync_copy(acc.at[pl.ds(row0 + mc * 125, 125)],
                                    rows1.at[pl.ds(0, 125)])

                    def add_body(j, _):
                        for h in range(2):
                            sl = pl.ds(h * 16, 16)
                            rows1[j, sl] = rows1[j, sl] + rows0[j, sl]
                        return 0
                    lax.fori_loop(0, 125, add_body, 0, unroll=4)
                    pltpu.sync_copy(
                        lay_hbm.at[pl.ds(r, 125), 1, pl.ds(c * HD, HD)],
                        rows0.at[pl.ds(0, 125)])

                    def fin_body(j, _):
                        for h in range(2):
                            sl = pl.ds(h * 16, 16)
                            rows1[j, sl] = (rows1[j, sl] + rows0[j, sl]) * (
                                1.0 / 3.0)
                        return 0
                    lax.fori_loop(0, 125, fin_body, 0, unroll=4)
                    pltpu.sync_copy(
                        rows1.at[pl.ds(0, 125)],
                        mean_hbm.at[pl.ds(r, 125), pl.ds(c * HD, HD)])
                    return 0
                lax.fori_loop(0, 25, mean_chunk, 0)

        @pl.when(s < 8)
        def _():
            flush_half(ulay_hbm, umean_hbm)

        @pl.when(s >= 8)
        def _():
            flush_half(ilay_hbm, imean_hbm)

        if layer < N_LAYERS - 1:
            pltpu.sync_copy(
                acc_sl, xbuf_hbm.at[pl.ds(c * NPAD + row0, ROWS_PER_TILE)])


@functools.lru_cache(maxsize=1)
def _sc_kernels():
    mesh = plsc.VectorSubcoreMesh(
        core_axis_name="c", subcore_axis_name="s",
        num_cores=2, num_subcores=16)
    sc_params = pltpu.CompilerParams(use_tc_tiling_on_sc=False)
    hop_gather = pl.kernel(
        _hop_gather_body,
        out_type=jax.ShapeDtypeStruct((HOP_TOTAL, D), jnp.float32),
        mesh=mesh,
        compiler_params=sc_params,
        scratch_types=[
            pltpu.VMEM((HOP_CHUNK,), jnp.int32),
            pltpu.VMEM((HOP_CHUNK, D), jnp.float32),
            pltpu.SemaphoreType.DMA,
        ],
    )
    spmm3 = pl.kernel(
        _spmm3_body,
        out_type=(
            jax.ShapeDtypeStruct((NUM_USERS, N_LAYERS, D), jnp.float32),
            jax.ShapeDtypeStruct((NUM_ITEMS, N_LAYERS, D), jnp.float32),
            jax.ShapeDtypeStruct((NUM_USERS, D), jnp.float32),
            jax.ShapeDtypeStruct((NUM_ITEMS, D), jnp.float32),
            jax.ShapeDtypeStruct((2 * NPAD, D // 2), jnp.float32),
        ),
        mesh=mesh,
        compiler_params=sc_params,
        scratch_types=[
            pltpu.VMEM_SHARED((NPAD, D // 2), jnp.float32),
            pltpu.VMEM((ECHUNK,), jnp.int32),
            pltpu.VMEM((ECHUNK,), jnp.int32),
            pltpu.VMEM((ECHUNK,), jnp.int32),
            pltpu.VMEM((ECHUNK,), jnp.int32),
            pltpu.VMEM((ECHUNK,), jnp.float32),
            pltpu.VMEM((ECHUNK,), jnp.float32),
            pltpu.VMEM((ECHUNK, D // 2), jnp.float32),
            pltpu.VMEM((ECHUNK, D // 2), jnp.float32),
            pltpu.SemaphoreType.DMA,
            pltpu.SemaphoreType.DMA,
            pltpu.SemaphoreType.DMA,
            pltpu.SemaphoreType.DMA,
            pltpu.SemaphoreType.DMA,
            pltpu.SemaphoreType.DMA,
        ],
    )
    return hop_gather, spmm3


def kernel(photo_one_hop, user_emb, item_emb, image_feats, text_feats,
           Q, K, V, W_onehop, W_mul1, W_mul2, edge_index, edge_weight):
    hop_gather, spmm3 = _sc_kernels()
    hop_idx = photo_one_hop.reshape(-1).astype(jnp.int32)
    hop_emb = hop_gather(hop_idx, user_emb)

    all_items = _dense_stage(item_emb, hop_emb, image_feats, text_feats,
                             Q, K, V, W_onehop, W_mul1, W_mul2)

    ego = jnp.concatenate([user_emb, all_items], axis=0)
    x0 = jnp.concatenate([ego[:, :D // 2], ego[:, D // 2:]], axis=0)
    zrows = jnp.zeros((ROWS_PER_TILE, D // 2), jnp.float32)

    src = edge_index[0].astype(jnp.int32)
    dst = edge_index[1].astype(jnp.int32)
    src2 = jnp.stack([src, src + NPAD], axis=0)
    ulay, ilay, umean, imean, _ = spmm3(x0, src2, dst, edge_weight, zrows)
    return (umean, imean, ulay, ilay)


# speedup vs baseline: 7.1639x; 1.0920x over previous
import functools

import jax
import jax.numpy as jnp
from jax import lax
from jax.experimental import pallas as pl
from jax.experimental.pallas import tpu as pltpu
from jax.experimental.pallas import tpu_sc as plsc

NUM_USERS = 25000
NUM_ITEMS = 25000
N = NUM_USERS + NUM_ITEMS
D = 64
N_LAYERS = 3
NH = 4
ATT = 16
L = 20
E = 800000

NPAD = 50000
ROWS_PER_TILE = NPAD // 16
EDGES_PER_TILE = E // 16
ECHUNK = 400
NCHUNK = EDGES_PER_TILE // ECHUNK

HOP_TOTAL = NUM_ITEMS * L
HOP_CHUNK = 1000
HOP_NCHUNK = HOP_TOTAL // HOP_CHUNK

def _hop_gather_body(idx_hbm, table_hbm, out_hbm, idx_v, rows_v, sem):
    c = lax.axis_index("c")
    s = lax.axis_index("s")
    wid = s * 2 + c
    for i in range(16):
        cid = wid + 32 * i

        @pl.when(cid < HOP_NCHUNK)
        def _():
            base = cid * HOP_CHUNK
            pltpu.sync_copy(idx_hbm.at[pl.ds(base, HOP_CHUNK)], idx_v)
            pltpu.async_copy(table_hbm.at[idx_v], rows_v, sem).wait()
            pltpu.sync_copy(rows_v, out_hbm.at[pl.ds(base, HOP_CHUNK)])


_RB = 200


def _dense_body(item_ref, hop_ref, img_ref, txt_ref, q_ref, k2_ref, v2_ref,
                wo_ref, w1_ref, w2_ref, out_ref):
    item = item_ref[...]
    hopp = hop_ref[...]
    q = jnp.dot(item, q_ref[...], preferred_element_type=jnp.float32)
    k = jnp.dot(hopp, k2_ref[...],
                preferred_element_type=jnp.float32).reshape(_RB, L // 2, 2 * D)
    v = jnp.dot(hopp, v2_ref[...],
                preferred_element_type=jnp.float32).reshape(_RB, L // 2, 2 * D)

    q128 = jnp.concatenate([q, q], axis=-1)
    p = q128[:, None, :] * k
    ai = lax.broadcasted_iota(jnp.int32, (2 * D, 2 * D), 0) // ATT
    li = lax.broadcasted_iota(jnp.int32, (2 * D, 2 * D), 1) // ATT
    mhead = jnp.where(ai == li, 0.125, 0.0)
    s = jnp.dot(p.reshape(_RB * (L // 2), 2 * D), mhead,
                preferred_element_type=jnp.float32).reshape(_RB, L // 2,
                                                            2 * D)
    m = jnp.max(s, axis=1, keepdims=True)
    mh = jnp.maximum(m[..., :D], m[..., D:])
    e = jnp.exp(s - jnp.concatenate([mh, mh], axis=-1))
    su = jnp.sum(e, axis=1, keepdims=True)
    sh = su[..., :D] + su[..., D:]
    att = e * (1.0 / jnp.concatenate([sh, sh], axis=-1))
    wsum = jnp.sum(att * v, axis=1)
    mha = wsum[:, :D] + wsum[:, D:]

    one_hop = jnp.dot(mha, wo_ref[...].T, preferred_element_type=jnp.float32)
    feats = jnp.concatenate([img_ref[...], txt_ref[...]], axis=-1)
    hidden = jnp.dot(feats, w1_ref[...].T, preferred_element_type=jnp.float32)
    itea = jnp.dot(hidden, w2_ref[...].T, preferred_element_type=jnp.float32)
    out_ref[...] = item * itea + one_hop


def _dense_stage(item_emb, hop_packed, image_feats, text_feats, Q, K, V,
                 W_onehop, W_mul1, W_mul2):
    z = jnp.zeros((D, D), jnp.float32)
    K2 = jnp.block([[K, z], [z, K]])
    V2 = jnp.block([[V, z], [z, V]])
    grid = NUM_ITEMS // _RB
    full = lambda shp: pl.BlockSpec(shp, lambda i: (0,) * len(shp))
    return pl.pallas_call(
        _dense_body,
        grid=(grid,),
        in_specs=[
            pl.BlockSpec((_RB, D), lambda i: (i, 0)),
            pl.BlockSpec((_RB * L // 2, 2 * D), lambda i: (i, 0)),
            pl.BlockSpec((_RB, 128), lambda i: (i, 0)),
            pl.BlockSpec((_RB, 128), lambda i: (i, 0)),
            full((D, NH * ATT)),
            full((2 * D, 2 * D)),
            full((2 * D, 2 * D)),
            full((D, D)),
            full((4 * D, 256)),
            full((D, 4 * D)),
        ],
        out_specs=pl.BlockSpec((_RB, D), lambda i: (i, 0)),
        out_shape=jax.ShapeDtypeStruct((NUM_ITEMS, D), jnp.float32),
        compiler_params=pltpu.CompilerParams(
            dimension_semantics=("parallel",)),
    )(item_emb, hop_packed, image_feats, text_feats, Q, K2, V2,
      W_onehop, W_mul1, W_mul2)


def _scale_rows(rows, wv):
    def grp_body(g, _):
        w16 = wv[pl.ds(g * 16, 16)]
        for kk in range(16):
            j = g * 16 + kk
            ws = lax.gather(
                w16, jnp.full((16, 1), kk, jnp.int32),
                lax.GatherDimensionNumbers(
                    offset_dims=(), collapsed_slice_dims=(0,),
                    start_index_map=(0,)),
                (1,), mode=lax.GatherScatterMode.PROMISE_IN_BOUNDS)
            a = rows[j, pl.ds(0, 16)] * ws
            b = rows[j, pl.ds(16, 16)] * ws
            rows[j, pl.ds(0, 16)] = a
            rows[j, pl.ds(16, 16)] = b
        return 0
    lax.fori_loop(0, ECHUNK // 16, grp_body, 0)


def _spmm3_body(x0_hbm, edge_hbm, w_hbm, zrows_hbm,
                ulay_hbm, ilay_hbm, umean_hbm, imean_hbm, xbuf_hbm,
                acc, sidx0, sidx1, didx0, didx1, sdidx0, sdidx1,
                wv0, wv1, rows0, rows1,
                si0, si1, sg0, sg1, ss0, ss1):
    c = lax.axis_index("c")
    s = lax.axis_index("s")
    HD = D // 2
    row0 = s * ROWS_PER_TILE
    ebase0 = s * EDGES_PER_TILE
    coff = c * NPAD

    def start_idx(i, sidx, didx, wv, sem):
        base = ebase0 + i * ECHUNK
        pltpu.async_copy(edge_hbm.at[0, pl.ds(base, ECHUNK)], sidx, sem)
        pltpu.async_copy(edge_hbm.at[1, pl.ds(base, ECHUNK)], didx, sem)
        pltpu.async_copy(w_hbm.at[pl.ds(base, ECHUNK)], wv, sem)

    def wait_idx(sidx, didx, wv, sem):
        pltpu.make_async_copy(edge_hbm.at[0, pl.ds(0, ECHUNK)], sidx,
                              sem).wait()
        pltpu.make_async_copy(edge_hbm.at[1, pl.ds(0, ECHUNK)], didx,
                              sem).wait()
        pltpu.make_async_copy(w_hbm.at[pl.ds(0, ECHUNK)], wv, sem).wait()

    def adj(sidx):
        def body(k, _):
            sl = pl.ds(k * 16, 16)
            sidx[sl] = sidx[sl] + coff
            return 0
        lax.fori_loop(0, ECHUNK // 16, body, 0, unroll=4)

    def wait_scatter(rows, didx, sem):
        pltpu.make_async_copy(rows, acc.at[didx], sem).wait()

    for layer in range(N_LAYERS):
        xsrc = x0_hbm if layer == 0 else xbuf_hbm

        pltpu.sync_copy(zrows_hbm, acc.at[pl.ds(row0, ROWS_PER_TILE)])
        plsc.subcore_barrier()

        def didx_save(didx, sdidx):
            def body(k, _):
                sl = pl.ds(k * 16, 16)
                sdidx[sl] = didx[sl]
                return 0
            lax.fori_loop(0, ECHUNK // 16, body, 0, unroll=4)

        start_idx(0, sidx0, didx0, wv0, si0)
        start_idx(1, sidx1, didx1, wv1, si1)
        wait_idx(sidx0, didx0, wv0, si0)
        adj(sidx0)
        didx_save(didx0, sdidx0)
        h0 = pltpu.async_copy(xsrc.at[sidx0], rows0, sg0)
        wait_idx(sidx1, didx1, wv1, si1)
        adj(sidx1)
        didx_save(didx1, sdidx1)
        h0.wait()
        h1 = pltpu.async_copy(xsrc.at[sidx1], rows1, sg1)
        _scale_rows(rows0, wv0)
        pltpu.async_copy(rows0, acc.at[sdidx0], ss0, add=True)
        start_idx(2, sidx0, didx0, wv0, si0)
        h1.wait()
        _scale_rows(rows1, wv1)
        pltpu.async_copy(rows1, acc.at[sdidx1], ss1, add=True)
        start_idx(3, sidx1, didx1, wv1, si1)

        def loop_body(g, _, xsrc=xsrc):
            wait_idx(sidx0, didx0, wv0, si0)
            adj(sidx0)
            wait_scatter(rows0, sdidx0, ss0)
            didx_save(didx0, sdidx0)
            ha = pltpu.async_copy(xsrc.at[sidx0], rows0, sg0)
            wait_idx(sidx1, didx1, wv1, si1)
            adj(sidx1)
            wait_scatter(rows1, sdidx1, ss1)
            didx_save(didx1, sdidx1)
            ha.wait()
            hb = pltpu.async_copy(xsrc.at[sidx1], rows1, sg1)
            _scale_rows(rows0, wv0)
            pltpu.async_copy(rows0, acc.at[sdidx0], ss0, add=True)
            start_idx(2 * g + 2, sidx0, didx0, wv0, si0)
            hb.wait()
            _scale_rows(rows1, wv1)
            pltpu.async_copy(rows1, acc.at[sdidx1], ss1, add=True)
            start_idx(jnp.minimum(2 * g + 3, NCHUNK - 1),
                      sidx1, didx1, wv1, si1)
            return 0

        lax.fori_loop(1, (NCHUNK - 1) // 2, loop_body, 0)
        wait_idx(sidx0, didx0, wv0, si0)
        adj(sidx0)
        wait_scatter(rows0, sdidx0, ss0)
        didx_save(didx0, sdidx0)
        hc = pltpu.async_copy(xsrc.at[sidx0], rows0, sg0)
        hc.wait()
        _scale_rows(rows0, wv0)
        pltpu.async_copy(rows0, acc.at[sdidx0], ss0, add=True)
        wait_scatter(rows1, sdidx1, ss1)
        wait_scatter(rows0, sdidx0, ss0)
        wait_idx(sidx1, didx1, wv1, si1)
        plsc.subcore_barrier()

        acc_sl = acc.at[pl.ds(row0, ROWS_PER_TILE)]
        lrow0 = jnp.where(s >= 8, row0 - NUM_USERS, row0)

        def flush_half(lay_hbm, mean_hbm):
            pltpu.sync_copy(
                acc_sl,
                lay_hbm.at[pl.ds(lrow0, ROWS_PER_TILE), layer,
                           pl.ds(c * HD, HD)])
            if layer == N_LAYERS - 1:
                def mean_chunk(mc, _):
                    r = lrow0 + mc * 125
                    pltpu.sync_copy(
                        lay_hbm.at[pl.ds(r, 125), 0, pl.ds(c * HD, HD)],
                        rows0.at[pl.ds(0, 125)])
                    pltpu.sync_copy(acc.at[pl.ds(row0 + mc * 125, 125)],
                                    rows1.at[pl.ds(0, 125)])

                    def add_body(j, _):
                        for h in range(2):
                            sl = pl.ds(h * 16, 16)
                            rows1[j, sl] = rows1[j, sl] + rows0[j, sl]
                        return 0
                    lax.fori_loop(0, 125, add_body, 0, unroll=4)
                    pltpu.sync_copy(
                        lay_hbm.at[pl.ds(r, 125), 1, pl.ds(c * HD, HD)],
                        rows0.at[pl.ds(0, 125)])

                    def fin_body(j, _):
                        for h in range(2):
                            sl = pl.ds(h * 16, 16)
                            rows1[j, sl] = (rows1[j, sl] + rows0[j, sl]) * (
                                1.0 / 3.0)
                        return 0
                    lax.fori_loop(0, 125, fin_body, 0, unroll=4)
                    pltpu.sync_copy(
                        rows1.at[pl.ds(0, 125)],
                        mean_hbm.at[pl.ds(r, 125), pl.ds(c * HD, HD)])
                    return 0
                lax.fori_loop(0, 25, mean_chunk, 0)

        @pl.when(s < 8)
        def _():
            flush_half(ulay_hbm, umean_hbm)

        @pl.when(s >= 8)
        def _():
            flush_half(ilay_hbm, imean_hbm)

        if layer < N_LAYERS - 1:
            pltpu.sync_copy(
                acc_sl, xbuf_hbm.at[pl.ds(c * NPAD + row0, ROWS_PER_TILE)])


@functools.lru_cache(maxsize=1)
def _sc_kernels():
    mesh = plsc.VectorSubcoreMesh(
        core_axis_name="c", subcore_axis_name="s",
        num_cores=2, num_subcores=16)
    sc_params = pltpu.CompilerParams(use_tc_tiling_on_sc=False)
    hop_gather = pl.kernel(
        _hop_gather_body,
        out_type=jax.ShapeDtypeStruct((HOP_TOTAL, D), jnp.float32),
        mesh=mesh,
        compiler_params=sc_params,
        scratch_types=[
            pltpu.VMEM((HOP_CHUNK,), jnp.int32),
            pltpu.VMEM((HOP_CHUNK, D), jnp.float32),
            pltpu.SemaphoreType.DMA,
        ],
    )
    spmm3 = pl.kernel(
        _spmm3_body,
        out_type=(
            jax.ShapeDtypeStruct((NUM_USERS, N_LAYERS, D), jnp.float32),
            jax.ShapeDtypeStruct((NUM_ITEMS, N_LAYERS, D), jnp.float32),
            jax.ShapeDtypeStruct((NUM_USERS, D), jnp.float32),
            jax.ShapeDtypeStruct((NUM_ITEMS, D), jnp.float32),
            jax.ShapeDtypeStruct((2 * NPAD, D // 2), jnp.float32),
        ),
        mesh=mesh,
        compiler_params=sc_params,
        scratch_types=[
            pltpu.VMEM_SHARED((NPAD, D // 2), jnp.float32),
            pltpu.VMEM((ECHUNK,), jnp.int32),
            pltpu.VMEM((ECHUNK,), jnp.int32),
            pltpu.VMEM((ECHUNK,), jnp.int32),
            pltpu.VMEM((ECHUNK,), jnp.int32),
            pltpu.VMEM((ECHUNK,), jnp.int32),
            pltpu.VMEM((ECHUNK,), jnp.int32),
            pltpu.VMEM((ECHUNK,), jnp.float32),
            pltpu.VMEM((ECHUNK,), jnp.float32),
            pltpu.VMEM((ECHUNK, D // 2), jnp.float32),
            pltpu.VMEM((ECHUNK, D // 2), jnp.float32),
            pltpu.SemaphoreType.DMA,
            pltpu.SemaphoreType.DMA,
            pltpu.SemaphoreType.DMA,
            pltpu.SemaphoreType.DMA,
            pltpu.SemaphoreType.DMA,
            pltpu.SemaphoreType.DMA,
        ],
    )
    return hop_gather, spmm3


def kernel(photo_one_hop, user_emb, item_emb, image_feats, text_feats,
           Q, K, V, W_onehop, W_mul1, W_mul2, edge_index, edge_weight):
    hop_gather, spmm3 = _sc_kernels()
    hop_idx = photo_one_hop.reshape(-1).astype(jnp.int32)
    hop_emb = hop_gather(hop_idx, user_emb).reshape(HOP_TOTAL // 2, 2 * D)

    all_items = _dense_stage(item_emb, hop_emb, image_feats, text_feats,
                             Q, K, V, W_onehop, W_mul1, W_mul2)

    ego = jnp.concatenate([user_emb, all_items], axis=0)
    x0 = jnp.concatenate([ego[:, :D // 2], ego[:, D // 2:]], axis=0)
    zrows = jnp.zeros((ROWS_PER_TILE, D // 2), jnp.float32)
    edges = edge_index.astype(jnp.int32)
    ulay, ilay, umean, imean, _ = spmm3(x0, edges, edge_weight, zrows)
    return (umean, imean, ulay, ilay)
